# baseline (speedup 1.0000x reference)
import functools
import jax
import jax.numpy as jnp
from jax import lax
from jax.experimental import pallas as pl
from jax.experimental.pallas import tpu as pltpu
from jax.experimental.pallas import tpu_sc as plsc

_NBUF = 4
_RPC = 4


@functools.partial(jax.jit, static_argnames=("b", "s"))
def _sc_gather(x1d, embedding, b, s):
    info = plsc.get_sparse_core_info()
    nc = info.num_cores
    nw = nc * info.num_subcores
    d = embedding.shape[1]
    rpw = b // nw
    n_chunks = rpw // _RPC
    cidx = _RPC * s
    ipw = rpw * s

    mesh = plsc.VectorSubcoreMesh(core_axis_name="c", subcore_axis_name="s")

    @functools.partial(
        pl.kernel,
        mesh=mesh,
        out_type=jax.ShapeDtypeStruct((b, s, d), jnp.float32),
        scratch_types=[
            pltpu.VMEM((ipw,), jnp.int32),
        ]
        + [pltpu.VMEM((cidx, d), jnp.float32)] * _NBUF
        + [pltpu.SemaphoreType.DMA] * (2 * _NBUF),
    )
    def k(x_hbm, tab_hbm, out_hbm, idx_all, *bufs):
        rows = bufs[:_NBUF]
        gsem = bufs[_NBUF : 2 * _NBUF]
        ssem = bufs[2 * _NBUF :]
        wid = lax.axis_index("s") * nc + lax.axis_index("c")
        base = wid * rpw

        pltpu.sync_copy(x_hbm.at[pl.ds(wid * ipw, ipw)], idx_all)

        def fire_gather(j, bf):
            pltpu.async_copy(
                tab_hbm.at[idx_all.at[pl.ds(j * cidx, cidx)]], rows[bf], gsem[bf]
            )

        def wait_gather(j, bf):
            pltpu.make_async_copy(
                tab_hbm.at[idx_all.at[pl.ds(j * cidx, cidx)]], rows[bf], gsem[bf]
            ).wait()

        def fire_scatters(j, bf):
            for i in range(_RPC):
                pltpu.async_copy(
                    rows[bf].at[pl.ds(i * s, s)],
                    out_hbm.at[base + j * _RPC + i],
                    ssem[bf],
                )

        def wait_scatters(j, bf):
            for i in range(_RPC):
                pltpu.make_async_copy(
                    rows[bf].at[pl.ds(i * s, s)],
                    out_hbm.at[base + j * _RPC + i],
                    ssem[bf],
                ).wait()

        for bf in range(_NBUF):
            fire_gather(bf, bf)

        wait_gather(0, 0)
        fire_scatters(0, 0)

        def outer(i, carry):
            for b in range(_NBUF):
                j = i * _NBUF + b + 1
                bf = (b + 1) % _NBUF
                pv = b % _NBUF
                wait_gather(j, bf)
                fire_scatters(j, bf)
                wait_scatters(j - 1, pv)
                fire_gather(j - 1 + _NBUF, pv)
            return carry

        lax.fori_loop(0, (n_chunks - _NBUF) // _NBUF, outer, 0)

        for b in range(_NBUF - 1):
            j = n_chunks - _NBUF + 1 + b
            wait_gather(j, j % _NBUF)
            fire_scatters(j, j % _NBUF)
            wait_scatters(j - 1, (j - 1) % _NBUF)
        wait_scatters(n_chunks - 1, (n_chunks - 1) % _NBUF)

    return k(x1d, embedding)


def kernel(X, embedding):
    b, s = X.shape
    info = plsc.get_sparse_core_info()
    nw = info.num_cores * info.num_subcores
    rpw = b // nw
    assert b == nw * rpw and rpw % (_RPC * _NBUF) == 0 and _RPC * s <= 128
    x1d = X.reshape(-1).astype(jnp.int32)
    return _sc_gather(x1d, embedding, b, s)


# speedup vs baseline: 1.0123x; 1.0123x over previous
import functools
import jax
import jax.numpy as jnp
from jax import lax
from jax.experimental import pallas as pl
from jax.experimental.pallas import tpu as pltpu
from jax.experimental.pallas import tpu_sc as plsc

_NBUF = 8
_RPC = 4


@functools.partial(jax.jit, static_argnames=("b", "s"))
def _sc_gather(x1d, embedding, b, s):
    info = plsc.get_sparse_core_info()
    nc = info.num_cores
    nw = nc * info.num_subcores
    d = embedding.shape[1]
    rpw = b // nw
    n_chunks = rpw // _RPC
    cidx = _RPC * s
    ipw = rpw * s

    mesh = plsc.VectorSubcoreMesh(core_axis_name="c", subcore_axis_name="s")

    @functools.partial(
        pl.kernel,
        mesh=mesh,
        out_type=jax.ShapeDtypeStruct((b, s, d), jnp.float32),
        scratch_types=[
            pltpu.VMEM((ipw,), jnp.int32),
        ]
        + [pltpu.VMEM((cidx, d), jnp.float32)] * _NBUF
        + [pltpu.SemaphoreType.DMA] * (2 * _NBUF),
    )
    def k(x_hbm, tab_hbm, out_hbm, idx_all, *bufs):
        rows = bufs[:_NBUF]
        gsem = bufs[_NBUF : 2 * _NBUF]
        ssem = bufs[2 * _NBUF :]
        wid = lax.axis_index("s") * nc + lax.axis_index("c")
        base = wid * rpw

        pltpu.sync_copy(x_hbm.at[pl.ds(wid * ipw, ipw)], idx_all)

        def fire_gather(j, bf):
            pltpu.async_copy(
                tab_hbm.at[idx_all.at[pl.ds(j * cidx, cidx)]], rows[bf], gsem[bf]
            )

        def wait_gather(j, bf):
            pltpu.make_async_copy(
                tab_hbm.at[idx_all.at[pl.ds(j * cidx, cidx)]], rows[bf], gsem[bf]
            ).wait()

        def fire_scatters(j, bf):
            for i in range(_RPC):
                pltpu.async_copy(
                    rows[bf].at[pl.ds(i * s, s)],
                    out_hbm.at[base + j * _RPC + i],
                    ssem[bf],
                )

        def wait_scatters(j, bf):
            for i in range(_RPC):
                pltpu.make_async_copy(
                    rows[bf].at[pl.ds(i * s, s)],
                    out_hbm.at[base + j * _RPC + i],
                    ssem[bf],
                ).wait()

        for bf in range(_NBUF):
            fire_gather(bf, bf)

        wait_gather(0, 0)
        fire_scatters(0, 0)

        def outer(i, carry):
            for b in range(_NBUF):
                j = i * _NBUF + b + 1
                bf = (b + 1) % _NBUF
                pv = b % _NBUF
                wait_gather(j, bf)
                fire_scatters(j, bf)
                wait_scatters(j - 1, pv)
                fire_gather(j - 1 + _NBUF, pv)
            return carry

        lax.fori_loop(0, (n_chunks - _NBUF) // _NBUF, outer, 0)

        for b in range(_NBUF - 1):
            j = n_chunks - _NBUF + 1 + b
            wait_gather(j, j % _NBUF)
            fire_scatters(j, j % _NBUF)
            wait_scatters(j - 1, (j - 1) % _NBUF)
        wait_scatters(n_chunks - 1, (n_chunks - 1) % _NBUF)

    return k(x1d, embedding)


def kernel(X, embedding):
    b, s = X.shape
    info = plsc.get_sparse_core_info()
    nw = info.num_cores * info.num_subcores
    rpw = b // nw
    assert b == nw * rpw and rpw % (_RPC * _NBUF) == 0 and _RPC * s <= 128
    x1d = X.reshape(-1).astype(jnp.int32)
    return _sc_gather(x1d, embedding, b, s)
